# baseline (speedup 1.0000x reference)
import functools

import jax
import jax.numpy as jnp
from jax import lax
from jax.experimental import pallas as pl
from jax.experimental.pallas import tpu as pltpu
from jax.experimental.pallas import tpu_sc as plsc

LABEL_SIZE = 1000
HIDDEN = 768
EPS = 1e-12

NUM_CORES = 2
NUM_SUBCORES = 16
NW = NUM_CORES * NUM_SUBCORES

ROW_CHUNKS = 8
CHUNK = 128
LAST_CHUNK = LABEL_SIZE - (ROW_CHUNKS - 1) * CHUNK
BATCH_GROUPS = NW // ROW_CHUNKS
WAVE = 16

TC_BATCH = 128
SC_BATCH = 128


def _ln_body(w_ref, gamma_ref, beta_ref, out_ref):
    x = w_ref[...]
    mu = jnp.mean(x, axis=-1, keepdims=True)
    var = jnp.mean(jnp.square(x - mu), axis=-1, keepdims=True)
    out_ref[...] = (x - mu) * lax.rsqrt(var + EPS) * gamma_ref[...] + beta_ref[...]


def _layer_norm_table(W, gamma, beta):
    return pl.pallas_call(
        _ln_body,
        out_shape=jax.ShapeDtypeStruct((LABEL_SIZE, HIDDEN), jnp.float32),
    )(W, gamma, beta)


def _tc_copy_body(ln_ref, out_ref):
    out_ref[...] = ln_ref[...][None, :, :]


def _tc_broadcast(ln, batch):
    return pl.pallas_call(
        _tc_copy_body,
        grid=(batch,),
        in_specs=[pl.BlockSpec((LABEL_SIZE, HIDDEN), lambda i: (0, 0))],
        out_specs=pl.BlockSpec((1, LABEL_SIZE, HIDDEN), lambda i: (i, 0, 0)),
        out_shape=jax.ShapeDtypeStruct((batch, LABEL_SIZE, HIDDEN), jnp.float32),
    )(ln)


def _stream_out(buf_slice, out_hbm, row0, nrows, b0, b_per_w, sem):
    for w0 in range(0, b_per_w, WAVE):
        nw = min(WAVE, b_per_w - w0)
        copies = [
            pltpu.async_copy(
                buf_slice, out_hbm.at[b0 + w0 + j, pl.ds(row0, nrows), :], sem
            )
            for j in range(nw)
        ]
        for c in copies:
            c.wait()


def _bcast_body(b_per_w, ln_hbm, out_hbm, buf, sem):
    wid = lax.axis_index("s") * NUM_CORES + lax.axis_index("c")
    rc = wid % ROW_CHUNKS
    bg = wid // ROW_CHUNKS
    row0 = rc * CHUNK
    b0 = bg * b_per_w

    @pl.when(rc < ROW_CHUNKS - 1)
    def _():
        pltpu.sync_copy(ln_hbm.at[pl.ds(row0, CHUNK), :], buf)
        _stream_out(buf, out_hbm, row0, CHUNK, b0, b_per_w, sem)

    @pl.when(rc == ROW_CHUNKS - 1)
    def _():
        small = buf.at[pl.ds(0, LAST_CHUNK), :]
        pltpu.sync_copy(ln_hbm.at[pl.ds(row0, LAST_CHUNK), :], small)
        _stream_out(small, out_hbm, row0, LAST_CHUNK, b0, b_per_w, sem)


def _sc_broadcast(ln, batch):
    assert batch % BATCH_GROUPS == 0
    b_per_w = batch // BATCH_GROUPS
    mesh = plsc.VectorSubcoreMesh(core_axis_name="c", subcore_axis_name="s")
    bcast = functools.partial(
        pl.kernel,
        out_type=jax.ShapeDtypeStruct((batch, LABEL_SIZE, HIDDEN), jnp.float32),
        mesh=mesh,
        scratch_types=[
            pltpu.VMEM((CHUNK, HIDDEN), jnp.float32),
            pltpu.SemaphoreType.DMA,
        ],
    )(functools.partial(_bcast_body, b_per_w))
    return bcast(ln)


def kernel(input_ids, W, gamma, beta):
    batch = input_ids.shape[0]
    assert batch == TC_BATCH + SC_BATCH

    ln = _layer_norm_table(W, gamma, beta)
    sc_part = _sc_broadcast(ln, SC_BATCH)
    tc_part = _tc_broadcast(ln, TC_BATCH)
    return jnp.concatenate([tc_part, sc_part], axis=0)


# speedup vs baseline: 2.8016x; 2.8016x over previous
import functools

import jax
import jax.numpy as jnp
from jax import lax
from jax.experimental import pallas as pl
from jax.experimental.pallas import tpu as pltpu
from jax.experimental.pallas import tpu_sc as plsc

LABEL_SIZE = 1000
HIDDEN = 768
EPS = 1e-12

NUM_CORES = 2
NUM_SUBCORES = 16
NW = NUM_CORES * NUM_SUBCORES

ROW_CHUNKS = 8
CHUNK = 128
LAST_CHUNK = LABEL_SIZE - (ROW_CHUNKS - 1) * CHUNK
BATCH_GROUPS = NW // ROW_CHUNKS
WAVE = 16


def _ln_body(w_ref, gamma_ref, beta_ref, out_ref):
    x = w_ref[...]
    mu = jnp.mean(x, axis=-1, keepdims=True)
    var = jnp.mean(jnp.square(x - mu), axis=-1, keepdims=True)
    out_ref[...] = (x - mu) * lax.rsqrt(var + EPS) * gamma_ref[...] + beta_ref[...]


def _layer_norm_table(W, gamma, beta):
    return pl.pallas_call(
        _ln_body,
        out_shape=jax.ShapeDtypeStruct((LABEL_SIZE, HIDDEN), jnp.float32),
    )(W, gamma, beta)


UNIT = 8
UNITS = LABEL_SIZE // UNIT
STAGE_UNITS = 5
INNER = 20


def _bcast_body(batch, ln_hbm, out_hbm, buf, sem):
    tasks_per_w = UNITS * batch // NW
    waves = tasks_per_w // INNER
    assert waves * INNER == tasks_per_w

    wid = lax.axis_index("s") * NUM_CORES + lax.axis_index("c")
    t0 = wid * tasks_per_w
    u0 = jnp.minimum(t0 // batch, UNITS - STAGE_UNITS)

    pltpu.sync_copy(ln_hbm.at[pl.ds(u0 * UNIT, STAGE_UNITS * UNIT), :], buf)

    def wave(i, _):
        base = t0 + i * INNER
        copies = []
        for j in range(INNER):
            g = base + j
            unit = g // batch
            b = g % batch
            local = unit - u0
            copies.append(
                pltpu.async_copy(
                    buf.at[pl.ds(local * UNIT, UNIT), :],
                    out_hbm.at[b, pl.ds(unit * UNIT, UNIT), :],
                    sem,
                )
            )
        for c in copies:
            c.wait()
        return _

    lax.fori_loop(0, waves, wave, None)


def kernel(input_ids, W, gamma, beta):
    batch = input_ids.shape[0]
    assert (UNITS * batch) % (NW * INNER) == 0

    ln = _layer_norm_table(W, gamma, beta)

    mesh = plsc.VectorSubcoreMesh(core_axis_name="c", subcore_axis_name="s")
    bcast = functools.partial(
        pl.kernel,
        out_type=jax.ShapeDtypeStruct((batch, LABEL_SIZE, HIDDEN), jnp.float32),
        mesh=mesh,
        scratch_types=[
            pltpu.VMEM((STAGE_UNITS * UNIT, HIDDEN), jnp.float32),
            pltpu.SemaphoreType.DMA,
        ],
    )(functools.partial(_bcast_body, batch))
    return bcast(ln)
